# baseline (speedup 1.0000x reference)
import functools

import jax
import jax.numpy as jnp
from jax import lax
from jax.experimental import pallas as pl
from jax.experimental.pallas import tpu as pltpu
from jax.experimental.pallas import tpu_sc as plsc

N = 10000
E = 320000
NP = 10112
STRIPE = NP // 16
EPAD = 327680
C = 128
NC, NS = 2, 16
F_DEG = 16
BLK = 512


def _mesh():
    return plsc.VectorSubcoreMesh(
        core_axis_name="c", subcore_axis_name="s", num_cores=NC, num_subcores=NS
    )


@functools.cache
def _segsum(F, b_out, Cg, with_deg=False):
    b_pc = b_out // NC
    K = 2 if with_deg else 4
    R = b_out * EPAD // Cg
    rpt = R // 32
    ngroups = rpt // K
    nfull, rem = STRIPE // Cg, STRIPE % Cg
    nwrit = nfull + (1 if rem else 0)

    def body(*refs):
        if with_deg:
            (x_hbm, src_hbm, dst_hbm, ones_hbm, out_hbm, deg_hbm,
             acc, acc_deg, isrc, idst, bufs, ones, gsem, ssem, isem) = refs
        else:
            (x_hbm, src_hbm, dst_hbm, out_hbm,
             acc, isrc, idst, bufs, gsem, ssem, isem) = refs
        cid = lax.axis_index("c")
        sid = lax.axis_index("s")
        wid = cid * NS + sid

        def zrow(r, _):
            for j in range(F // 16):
                bufs[0, r, pl.ds(j * 16, 16)] = jnp.zeros((16,), jnp.float32)
            return 0

        lax.fori_loop(0, Cg, zrow, 0)
        if with_deg:
            def zdrow(r, _):
                ones[r, :] = jnp.zeros((16,), jnp.float32)
                return 0

            lax.fori_loop(0, Cg, zdrow, 0)
        zd = []
        for l in range(b_pc):
            base = l * NP + sid * STRIPE
            for k in range(nwrit):
                cnt = Cg if k < nfull else rem
                sl = pl.ds(pl.multiple_of(base + k * Cg, 8), cnt)
                zd.append(pltpu.async_copy(
                    bufs.at[0, pl.ds(0, cnt)], acc.at[sl], isem))
                if with_deg:
                    zd.append(pltpu.async_copy(
                        ones.at[pl.ds(0, cnt)], acc_deg.at[sl], isem))
        for d in zd:
            d.wait()
        if with_deg:
            pltpu.sync_copy(ones_hbm, ones)
        plsc.subcore_barrier()

        row0 = wid * rpt

        def idx_rows(g):
            return pl.ds(pl.multiple_of(row0 + g * K, K), K)

        pltpu.sync_copy(src_hbm.at[idx_rows(0)], isrc.at[0])
        pltpu.sync_copy(dst_hbm.at[idx_rows(0)], idst.at[0])
        for k in range(K):
            pltpu.async_copy(x_hbm.at[isrc.at[0, k]], bufs.at[k], gsem)
        pltpu.async_copy(src_hbm.at[idx_rows(1)], isrc.at[1], isem)
        pltpu.async_copy(dst_hbm.at[idx_rows(1)], idst.at[1], isem)

        def quad(i, _):
            for j in range(4):
                S = j % 2
                T = 1 - S
                I = j
                g = 4 * i + j
                for k in range(K):
                    pltpu.make_async_copy(
                        x_hbm.at[pl.ds(0, Cg)], bufs.at[S * K + k], gsem).wait()
                for k in range(K):
                    pltpu.async_copy(bufs.at[S * K + k],
                                     acc.at[idst.at[I, k]], ssem, add=True)
                    if with_deg:
                        pltpu.async_copy(ones, acc_deg.at[idst.at[I, k]],
                                         ssem, add=True)

                @pl.when(g + 1 < ngroups)
                def _():
                    pltpu.make_async_copy(
                        src_hbm.at[pl.ds(0, K)], isrc.at[(j + 1) % 4], isem).wait()
                    pltpu.make_async_copy(
                        dst_hbm.at[pl.ds(0, K)], idst.at[(j + 1) % 4], isem).wait()

                @pl.when(jnp.logical_and(g >= 1, g + 1 < ngroups))
                def _():
                    for k in range(K):
                        pltpu.make_async_copy(
                            x_hbm.at[pl.ds(0, Cg)], bufs.at[T * K + k], ssem).wait()
                        if with_deg:
                            pltpu.make_async_copy(
                                ones_hbm, ones, ssem).wait()

                @pl.when(g + 1 < ngroups)
                def _():
                    for k in range(K):
                        pltpu.async_copy(x_hbm.at[isrc.at[(j + 1) % 4, k]],
                                         bufs.at[T * K + k], gsem)

                @pl.when(g + 2 < ngroups)
                def _():
                    pltpu.async_copy(src_hbm.at[idx_rows(g + 2)],
                                     isrc.at[(j + 2) % 4], isem)
                    pltpu.async_copy(dst_hbm.at[idx_rows(g + 2)],
                                     idst.at[(j + 2) % 4], isem)
            return 0

        lax.fori_loop(0, ngroups // 4, quad, 0)
        for _ in range(2 * K):
            pltpu.make_async_copy(
                x_hbm.at[pl.ds(0, Cg)], bufs.at[0], ssem).wait()
            if with_deg:
                pltpu.make_async_copy(ones_hbm, ones, ssem).wait()
        plsc.subcore_barrier()

        wd = []
        for l in range(b_pc):
            sbase = l * NP + sid * STRIPE
            obase = (cid * b_pc + l) * NP + sid * STRIPE
            for k in range(nwrit):
                cnt = Cg if k < nfull else rem
                ssl = pl.ds(pl.multiple_of(sbase + k * Cg, 8), cnt)
                osl = pl.ds(pl.multiple_of(obase + k * Cg, 8), cnt)
                wd.append(pltpu.async_copy(acc.at[ssl], out_hbm.at[osl], isem))
                if with_deg:
                    wd.append(pltpu.async_copy(
                        acc_deg.at[ssl], deg_hbm.at[osl], isem))
        for d in wd:
            d.wait()

    out_type = jax.ShapeDtypeStruct((b_out * NP, F), jnp.float32)
    scratch = [
        pltpu.VMEM_SHARED((b_pc * NP, F), jnp.float32),
        pltpu.VMEM((4, K, Cg), jnp.int32),
        pltpu.VMEM((4, K, Cg), jnp.int32),
        pltpu.VMEM((2 * K, Cg, F), jnp.float32),
        pltpu.SemaphoreType.DMA,
        pltpu.SemaphoreType.DMA,
        pltpu.SemaphoreType.DMA,
    ]
    if with_deg:
        out_type = [out_type,
                    jax.ShapeDtypeStruct((b_out * NP, F_DEG), jnp.float32)]
        scratch.insert(1, pltpu.VMEM_SHARED((b_pc * NP, F_DEG), jnp.float32))
        scratch.insert(5, pltpu.VMEM((Cg, F_DEG), jnp.float32))
    return pl.kernel(
        body,
        out_type=out_type,
        mesh=_mesh(),
        compiler_params=pltpu.CompilerParams(use_tc_tiling_on_sc=False),
        scratch_types=scratch,
    )


_NBLK = pl.cdiv(N, BLK)


def _matmul(x, w, bias=None):
    B, n, Fi = x.shape
    Fo = w.shape[2]

    def body(x_ref, w_ref, *rest):
        if bias is not None:
            b_ref, o_ref = rest
        else:
            (o_ref,) = rest
        r = jnp.dot(x_ref[0], w_ref[0], preferred_element_type=jnp.float32)
        if bias is not None:
            r = r + b_ref[0]
        o_ref[0] = r

    in_specs = [
        pl.BlockSpec((1, BLK, Fi), lambda b, i: (b, i, 0)),
        pl.BlockSpec((1, Fi, Fo), lambda b, i: (b, 0, 0)),
    ]
    args = [x, w]
    if bias is not None:
        in_specs.append(pl.BlockSpec((1, 1, Fo), lambda b, i: (b, 0, 0)))
        args.append(bias.reshape(B, 1, Fo))
    return pl.pallas_call(
        body,
        grid=(B, _NBLK),
        in_specs=in_specs,
        out_specs=pl.BlockSpec((1, BLK, Fo), lambda b, i: (b, i, 0)),
        out_shape=jax.ShapeDtypeStruct((B, n, Fo), jnp.float32),
    )(*args)


def _combine(msg, y, dup, deg, bias):
    B, _, F = msg.shape

    def body(m_ref, y_ref, d_ref, *rest):
        if bias is not None:
            b_ref, o_ref = rest
        else:
            (o_ref,) = rest
        inv = 1.0 / (d_ref[0][:, 0:1] + 1.0)
        r = (m_ref[0] + y_ref[0]) * inv
        if bias is not None:
            r = r + b_ref[0]
        o_ref[0] = r

    in_specs = [
        pl.BlockSpec((1, BLK, F), lambda b, i: (b, i, 0)),
        pl.BlockSpec((1, BLK, F), lambda b, i: (b // dup, i, 0)),
        pl.BlockSpec((1, BLK, F_DEG), lambda b, i: (b, i, 0)),
    ]
    args = [msg, y, deg]
    if bias is not None:
        in_specs.append(pl.BlockSpec((1, 1, F), lambda b, i: (b, 0, 0)))
        args.append(bias.reshape(B, 1, F))
    return pl.pallas_call(
        body,
        grid=(B, _NBLK),
        in_specs=in_specs,
        out_specs=pl.BlockSpec((1, BLK, F), lambda b, i: (b, i, 0)),
        out_shape=jax.ShapeDtypeStruct((B, N, F), jnp.float32),
    )(*args)


def _cmm(msg, y, dup, deg, bias_c, W, bias_m, W2, emit_x):
    B, _, F = msg.shape
    Fo = W.shape[2]
    Ft = W2.shape[2] if W2 is not None else Fo

    def body(*refs):
        it = iter(refs)
        m_ref = next(it)
        y_ref = next(it)
        d_ref = next(it)
        bc_ref = next(it) if bias_c is not None else None
        w_ref = next(it)
        bm_ref = next(it) if bias_m is not None else None
        w2_ref = next(it) if W2 is not None else None
        x_ref = next(it) if emit_x else None
        t_ref = next(it)
        inv = 1.0 / (d_ref[0][:, 0:1] + 1.0)
        x = (m_ref[0] + y_ref[0]) * inv
        if bias_c is not None:
            x = x + bc_ref[0]
        t = jnp.dot(x, w_ref[0], preferred_element_type=jnp.float32)
        if bias_m is not None:
            t = t + bm_ref[0]
        if W2 is not None:
            t = jnp.dot(t, w2_ref[0], preferred_element_type=jnp.float32)
        if emit_x:
            x_ref[0] = x
        t_ref[0] = t

    in_specs = [
        pl.BlockSpec((1, BLK, F), lambda b, i: (b, i, 0)),
        pl.BlockSpec((1, BLK, F), lambda b, i: (b // dup, i, 0)),
        pl.BlockSpec((1, BLK, F_DEG), lambda b, i: (b, i, 0)),
    ]
    args = [msg, y, deg]
    if bias_c is not None:
        in_specs.append(pl.BlockSpec((1, 1, F), lambda b, i: (b, 0, 0)))
        args.append(bias_c.reshape(B, 1, F))
    in_specs.append(pl.BlockSpec((1, F, Fo), lambda b, i: (b, 0, 0)))
    args.append(W)
    if bias_m is not None:
        in_specs.append(pl.BlockSpec((1, 1, Fo), lambda b, i: (b, 0, 0)))
        args.append(bias_m.reshape(B, 1, Fo))
    if W2 is not None:
        in_specs.append(pl.BlockSpec((1, Fo, Ft), lambda b, i: (b, 0, 0)))
        args.append(W2)
    out_specs = []
    out_shape = []
    if emit_x:
        out_specs.append(pl.BlockSpec((1, BLK, F), lambda b, i: (b, i, 0)))
        out_shape.append(jax.ShapeDtypeStruct((B, N, F), jnp.float32))
    out_specs.append(pl.BlockSpec((1, BLK, Ft), lambda b, i: (b, i, 0)))
    out_shape.append(jax.ShapeDtypeStruct((B, N, Ft), jnp.float32))
    r = pl.pallas_call(
        body,
        grid=(B, _NBLK),
        in_specs=in_specs,
        out_specs=out_specs,
        out_shape=out_shape,
    )(*args)
    return r if emit_x else r[0]


def _attn_core(e1, e2, w_om, u_om):
    v1 = jnp.tanh(jnp.dot(e1, w_om, preferred_element_type=jnp.float32))
    u1 = jnp.dot(v1, u_om, preferred_element_type=jnp.float32)
    v2 = jnp.tanh(jnp.dot(e2, w_om, preferred_element_type=jnp.float32))
    u2 = jnp.dot(v2, u_om, preferred_element_type=jnp.float32)
    m = jnp.maximum(u1, u2)
    a1 = jnp.exp(u1 - m)
    a2 = jnp.exp(u2 - m)
    s = a1 + a2
    a1 = a1 / s
    a2 = a2 / s
    return a1 * e1 + a2 * e2, jnp.concatenate([a1, a2], axis=1)


def _attn_prop(xs_sp, xs_fe, wt_sp, wt_fe, w_om, u_om):
    H = xs_sp.shape[2]

    def body(xs_ref, xf_ref, ws_ref, wf_ref, w_ref, u_ref, lat_ref, al_ref):
        ws = ws_ref[...]
        ws = jnp.exp(ws - jnp.max(ws))
        ws = ws / jnp.sum(ws)
        wf = wf_ref[...]
        wf = jnp.exp(wf - jnp.max(wf))
        wf = wf / jnp.sum(wf)
        e1 = ws[0] * xs_ref[0] + ws[1] * xs_ref[1] + ws[2] * xs_ref[2]
        e2 = wf[0] * xf_ref[0] + wf[1] * xf_ref[1] + wf[2] * xf_ref[2]
        lat, al = _attn_core(e1, e2, w_ref[...], u_ref[...])
        lat_ref[...] = lat
        al_ref[...] = al

    return pl.pallas_call(
        body,
        grid=(_NBLK,),
        in_specs=[
            pl.BlockSpec((3, BLK, H), lambda i: (0, i, 0)),
            pl.BlockSpec((3, BLK, H), lambda i: (0, i, 0)),
            pl.BlockSpec((3,), lambda i: (0,)),
            pl.BlockSpec((3,), lambda i: (0,)),
            pl.BlockSpec((H, H), lambda i: (0, 0)),
            pl.BlockSpec((H, 1), lambda i: (0, 0)),
        ],
        out_specs=[
            pl.BlockSpec((BLK, H), lambda i: (i, 0)),
            pl.BlockSpec((BLK, 2), lambda i: (i, 0)),
        ],
        out_shape=[
            jax.ShapeDtypeStruct((N, H), jnp.float32),
            jax.ShapeDtypeStruct((N, 2), jnp.float32),
        ],
    )(xs_sp, xs_fe, wt_sp, wt_fe, w_om, u_om)


def _attn_pair(e1, e2, w_om, u_om):
    H = e1.shape[1]

    def body(e1_ref, e2_ref, w_ref, u_ref, lat_ref, al_ref):
        lat, al = _attn_core(e1_ref[...], e2_ref[...], w_ref[...], u_ref[...])
        lat_ref[...] = lat
        al_ref[...] = al

    return pl.pallas_call(
        body,
        grid=(_NBLK,),
        in_specs=[
            pl.BlockSpec((BLK, H), lambda i: (i, 0)),
            pl.BlockSpec((BLK, H), lambda i: (i, 0)),
            pl.BlockSpec((H, H), lambda i: (0, 0)),
            pl.BlockSpec((H, 1), lambda i: (0, 0)),
        ],
        out_specs=[
            pl.BlockSpec((BLK, H), lambda i: (i, 0)),
            pl.BlockSpec((BLK, 2), lambda i: (i, 0)),
        ],
        out_shape=[
            jax.ShapeDtypeStruct((N, H), jnp.float32),
            jax.ShapeDtypeStruct((N, 2), jnp.float32),
        ],
    )(e1, e2, w_om, u_om)


def kernel(edge_spatial_omics1, edge_feature_omics1, feat_omics1,
           edge_spatial_omics2, edge_feature_omics2, feat_omics2,
           enc1, enc2, dec1, dec2, a1_w, a1_u, a2_w, a2_u, ac_w, ac_u,
           wt1, wt2, wt3, wt4):
    def pad(e):
        src = jnp.concatenate(
            [e[0].astype(jnp.int32), jnp.zeros((EPAD - E,), jnp.int32)])
        dst = jnp.concatenate(
            [e[1].astype(jnp.int32), jnp.full((EPAD - E,), N, jnp.int32)])
        return src, dst

    s1, d1 = pad(edge_spatial_omics1)
    s2, d2 = pad(edge_feature_omics1)
    s3, d3 = pad(edge_spatial_omics2)
    s4, d4 = pad(edge_feature_omics2)
    dst4_c = jnp.concatenate([d1, d2 + NP, d3, d4 + NP])
    dst4_64 = dst4_c.reshape(-1, 64)
    src4_2in = jnp.concatenate([s1, s2, s3 + N, s4 + N]).reshape(-1, 64)
    src4_4in = jnp.concatenate(
        [s1, s2 + N, s3 + 2 * N, s4 + 3 * N]).reshape(-1, 64)
    dst2_c = jnp.concatenate([d1, d3])
    dst2_64 = dst2_c.reshape(-1, 64)
    dst2_32 = dst2_c.reshape(-1, 32)
    src2_1in = jnp.concatenate([s1, s3]).reshape(-1, 64)
    src2_2in = jnp.concatenate([s1, s3 + N]).reshape(-1, 32)

    ones = jnp.ones((64, F_DEG), jnp.float32)

    W1 = jnp.stack([enc1[0][0], enc2[0][0]])
    y = _matmul(jnp.stack([feat_omics1, feat_omics2]), W1)
    msg, deg4 = _segsum(64, 4, 64, True)(
        y.reshape(2 * N, 64), src4_2in, dst4_64, ones)
    deg4 = deg4.reshape(4, NP, F_DEG)
    deg_dec = jnp.stack([deg4[0], deg4[2]])
    b1 = jnp.stack([enc1[0][1], enc1[0][1], enc2[0][1], enc2[0][1]])
    W_l = [jnp.stack([enc1[l][0], enc1[l][0], enc2[l][0], enc2[l][0]])
           for l in (1, 2)]
    b_l = [jnp.stack([enc1[l][1], enc1[l][1], enc2[l][1], enc2[l][1]])
           for l in (1, 2)]
    x, y = _cmm(msg.reshape(4, NP, 64), y, 2, deg4, b1, W_l[0], None, None, True)
    hcell = [x]
    msg = _segsum(64, 4, 64)(y.reshape(4 * N, 64), src4_4in, dst4_64)
    x, y = _cmm(msg.reshape(4, NP, 64), y, 1, deg4, b_l[0], W_l[1], None, None, True)
    hcell.append(x)
    msg = _segsum(64, 4, 64)(y.reshape(4 * N, 64), src4_4in, dst4_64)
    x = _combine(msg.reshape(4, NP, 64), y, 1, deg4, b_l[1])
    hcell.append(x)

    xs_sp1 = jnp.stack([h[0] for h in hcell])
    xs_fe1 = jnp.stack([h[1] for h in hcell])
    xs_sp2 = jnp.stack([h[2] for h in hcell])
    xs_fe2 = jnp.stack([h[3] for h in hcell])

    lat1, _ = _attn_prop(xs_sp1, xs_fe1, wt1, wt2, a1_w, a1_u)
    lat2, _ = _attn_prop(xs_sp2, xs_fe2, wt3, wt4, a2_w, a2_u)
    combined, alpha_cross = _attn_pair(lat1, lat2, ac_w, ac_u)

    msg = _segsum(64, 2, 64)(combined, src2_1in, dst2_64).reshape(2, NP, 64)
    Wd = [jnp.stack([dec1[l][0], dec2[l][0]]) for l in (0, 1, 2)]
    bd = [jnp.stack([dec1[l][1], dec2[l][1]]) for l in (0, 1, 2)]
    yd = _cmm(msg, combined.reshape(1, N, 64), 2, deg_dec,
              None, Wd[0], bd[0], Wd[1], False)
    msg = _segsum(128, 2, 32)(yd.reshape(2 * N, 128), src2_2in, dst2_32)
    yd = _cmm(msg.reshape(2, NP, 128), yd, 1, deg_dec, bd[1], Wd[2],
              None, None, False)
    msg = _segsum(128, 2, 32)(yd.reshape(2 * N, 128), src2_2in, dst2_32)
    xd = _combine(msg.reshape(2, NP, 128), yd, 1, deg_dec, bd[2])

    return (lat1, lat2, combined, xd[0], xd[1], alpha_cross)


# speedup vs baseline: 1.0373x; 1.0373x over previous
import functools

import jax
import jax.numpy as jnp
from jax import lax
from jax.experimental import pallas as pl
from jax.experimental.pallas import tpu as pltpu
from jax.experimental.pallas import tpu_sc as plsc

N = 10000
E = 320000
NP = 10112
STRIPE = NP // 16
EPAD = 327680
C = 128
NC, NS = 2, 16
F_DEG = 16
BLK = 512


def _mesh():
    return plsc.VectorSubcoreMesh(
        core_axis_name="c", subcore_axis_name="s", num_cores=NC, num_subcores=NS
    )


@functools.cache
def _segsum(F, b_out, Cg, with_deg=False):
    b_pc = b_out // NC
    if with_deg:
        K = 2
    elif b_out == 2 and F == 64:
        K = 8
    else:
        K = 4
    R = b_out * EPAD // Cg
    rpt = R // 32
    ngroups = rpt // K
    nfull, rem = STRIPE // Cg, STRIPE % Cg
    nwrit = nfull + (1 if rem else 0)

    def body(*refs):
        if with_deg:
            (x_hbm, src_hbm, dst_hbm, ones_hbm, out_hbm, deg_hbm,
             acc, acc_deg, isrc, idst, bufs, ones, gsem, ssem, isem) = refs
        else:
            (x_hbm, src_hbm, dst_hbm, out_hbm,
             acc, isrc, idst, bufs, gsem, ssem, isem) = refs
        cid = lax.axis_index("c")
        sid = lax.axis_index("s")
        wid = cid * NS + sid

        def zrow(r, _):
            for j in range(F // 16):
                bufs[0, r, pl.ds(j * 16, 16)] = jnp.zeros((16,), jnp.float32)
            return 0

        lax.fori_loop(0, Cg, zrow, 0)
        if with_deg:
            def zdrow(r, _):
                ones[r, :] = jnp.zeros((16,), jnp.float32)
                return 0

            lax.fori_loop(0, Cg, zdrow, 0)
        zd = []
        for l in range(b_pc):
            base = l * NP + sid * STRIPE
            for k in range(nwrit):
                cnt = Cg if k < nfull else rem
                sl = pl.ds(pl.multiple_of(base + k * Cg, 8), cnt)
                zd.append(pltpu.async_copy(
                    bufs.at[0, pl.ds(0, cnt)], acc.at[sl], isem))
                if with_deg:
                    zd.append(pltpu.async_copy(
                        ones.at[pl.ds(0, cnt)], acc_deg.at[sl], isem))
        for d in zd:
            d.wait()
        if with_deg:
            pltpu.sync_copy(ones_hbm, ones)
        plsc.subcore_barrier()

        row0 = wid * rpt

        def idx_rows(g):
            return pl.ds(pl.multiple_of(row0 + g * K, K), K)

        pltpu.sync_copy(src_hbm.at[idx_rows(0)], isrc.at[0])
        pltpu.sync_copy(dst_hbm.at[idx_rows(0)], idst.at[0])
        for k in range(K):
            pltpu.async_copy(x_hbm.at[isrc.at[0, k]], bufs.at[k], gsem)
        pltpu.async_copy(src_hbm.at[idx_rows(1)], isrc.at[1], isem)
        pltpu.async_copy(dst_hbm.at[idx_rows(1)], idst.at[1], isem)

        def quad(i, _):
            for j in range(4):
                S = j % 2
                T = 1 - S
                I = j
                g = 4 * i + j
                for k in range(K):
                    pltpu.make_async_copy(
                        x_hbm.at[pl.ds(0, Cg)], bufs.at[S * K + k], gsem).wait()
                for k in range(K):
                    pltpu.async_copy(bufs.at[S * K + k],
                                     acc.at[idst.at[I, k]], ssem, add=True)
                    if with_deg:
                        pltpu.async_copy(ones, acc_deg.at[idst.at[I, k]],
                                         ssem, add=True)

                @pl.when(g + 1 < ngroups)
                def _():
                    pltpu.make_async_copy(
                        src_hbm.at[pl.ds(0, K)], isrc.at[(j + 1) % 4], isem).wait()
                    pltpu.make_async_copy(
                        dst_hbm.at[pl.ds(0, K)], idst.at[(j + 1) % 4], isem).wait()

                @pl.when(jnp.logical_and(g >= 1, g + 1 < ngroups))
                def _():
                    for k in range(K):
                        pltpu.make_async_copy(
                            x_hbm.at[pl.ds(0, Cg)], bufs.at[T * K + k], ssem).wait()
                        if with_deg:
                            pltpu.make_async_copy(
                                ones_hbm, ones, ssem).wait()

                @pl.when(g + 1 < ngroups)
                def _():
                    for k in range(K):
                        pltpu.async_copy(x_hbm.at[isrc.at[(j + 1) % 4, k]],
                                         bufs.at[T * K + k], gsem)

                @pl.when(g + 2 < ngroups)
                def _():
                    pltpu.async_copy(src_hbm.at[idx_rows(g + 2)],
                                     isrc.at[(j + 2) % 4], isem)
                    pltpu.async_copy(dst_hbm.at[idx_rows(g + 2)],
                                     idst.at[(j + 2) % 4], isem)
            return 0

        lax.fori_loop(0, ngroups // 4, quad, 0)
        for _ in range(2 * K):
            pltpu.make_async_copy(
                x_hbm.at[pl.ds(0, Cg)], bufs.at[0], ssem).wait()
            if with_deg:
                pltpu.make_async_copy(ones_hbm, ones, ssem).wait()
        plsc.subcore_barrier()

        wd = []
        for l in range(b_pc):
            sbase = l * NP + sid * STRIPE
            obase = (cid * b_pc + l) * NP + sid * STRIPE
            for k in range(nwrit):
                cnt = Cg if k < nfull else rem
                ssl = pl.ds(pl.multiple_of(sbase + k * Cg, 8), cnt)
                osl = pl.ds(pl.multiple_of(obase + k * Cg, 8), cnt)
                wd.append(pltpu.async_copy(acc.at[ssl], out_hbm.at[osl], isem))
                if with_deg:
                    wd.append(pltpu.async_copy(
                        acc_deg.at[ssl], deg_hbm.at[osl], isem))
        for d in wd:
            d.wait()

    out_type = jax.ShapeDtypeStruct((b_out * NP, F), jnp.float32)
    scratch = [
        pltpu.VMEM_SHARED((b_pc * NP, F), jnp.float32),
        pltpu.VMEM((4, K, Cg), jnp.int32),
        pltpu.VMEM((4, K, Cg), jnp.int32),
        pltpu.VMEM((2 * K, Cg, F), jnp.float32),
        pltpu.SemaphoreType.DMA,
        pltpu.SemaphoreType.DMA,
        pltpu.SemaphoreType.DMA,
    ]
    if with_deg:
        out_type = [out_type,
                    jax.ShapeDtypeStruct((b_out * NP, F_DEG), jnp.float32)]
        scratch.insert(1, pltpu.VMEM_SHARED((b_pc * NP, F_DEG), jnp.float32))
        scratch.insert(5, pltpu.VMEM((Cg, F_DEG), jnp.float32))
    return pl.kernel(
        body,
        out_type=out_type,
        mesh=_mesh(),
        compiler_params=pltpu.CompilerParams(use_tc_tiling_on_sc=False),
        scratch_types=scratch,
    )


_NBLK = pl.cdiv(N, BLK)


def _matmul(x, w, bias=None):
    B, n, Fi = x.shape
    Fo = w.shape[2]

    def body(x_ref, w_ref, *rest):
        if bias is not None:
            b_ref, o_ref = rest
        else:
            (o_ref,) = rest
        r = jnp.dot(x_ref[0], w_ref[0], preferred_element_type=jnp.float32)
        if bias is not None:
            r = r + b_ref[0]
        o_ref[0] = r

    in_specs = [
        pl.BlockSpec((1, BLK, Fi), lambda b, i: (b, i, 0)),
        pl.BlockSpec((1, Fi, Fo), lambda b, i: (b, 0, 0)),
    ]
    args = [x, w]
    if bias is not None:
        in_specs.append(pl.BlockSpec((1, 1, Fo), lambda b, i: (b, 0, 0)))
        args.append(bias.reshape(B, 1, Fo))
    return pl.pallas_call(
        body,
        grid=(B, _NBLK),
        in_specs=in_specs,
        out_specs=pl.BlockSpec((1, BLK, Fo), lambda b, i: (b, i, 0)),
        out_shape=jax.ShapeDtypeStruct((B, n, Fo), jnp.float32),
    )(*args)


def _combine(msg, y, dup, deg, bias):
    B, _, F = msg.shape

    def body(m_ref, y_ref, d_ref, *rest):
        if bias is not None:
            b_ref, o_ref = rest
        else:
            (o_ref,) = rest
        inv = 1.0 / (d_ref[0][:, 0:1] + 1.0)
        r = (m_ref[0] + y_ref[0]) * inv
        if bias is not None:
            r = r + b_ref[0]
        o_ref[0] = r

    in_specs = [
        pl.BlockSpec((1, BLK, F), lambda b, i: (b, i, 0)),
        pl.BlockSpec((1, BLK, F), lambda b, i: (b // dup, i, 0)),
        pl.BlockSpec((1, BLK, F_DEG), lambda b, i: (b, i, 0)),
    ]
    args = [msg, y, deg]
    if bias is not None:
        in_specs.append(pl.BlockSpec((1, 1, F), lambda b, i: (b, 0, 0)))
        args.append(bias.reshape(B, 1, F))
    return pl.pallas_call(
        body,
        grid=(B, _NBLK),
        in_specs=in_specs,
        out_specs=pl.BlockSpec((1, BLK, F), lambda b, i: (b, i, 0)),
        out_shape=jax.ShapeDtypeStruct((B, N, F), jnp.float32),
    )(*args)


def _attn_core(e1, e2, w_om, u_om):
    v1 = jnp.tanh(jnp.dot(e1, w_om, preferred_element_type=jnp.float32))
    u1 = jnp.dot(v1, u_om, preferred_element_type=jnp.float32)
    v2 = jnp.tanh(jnp.dot(e2, w_om, preferred_element_type=jnp.float32))
    u2 = jnp.dot(v2, u_om, preferred_element_type=jnp.float32)
    m = jnp.maximum(u1, u2)
    a1 = jnp.exp(u1 - m)
    a2 = jnp.exp(u2 - m)
    s = a1 + a2
    a1 = a1 / s
    a2 = a2 / s
    return a1 * e1 + a2 * e2, jnp.concatenate([a1, a2], axis=1)


def _attn_prop(xs_sp, xs_fe, wt_sp, wt_fe, w_om, u_om):
    H = xs_sp.shape[2]

    def body(xs_ref, xf_ref, ws_ref, wf_ref, w_ref, u_ref, lat_ref, al_ref):
        ws = ws_ref[...]
        ws = jnp.exp(ws - jnp.max(ws))
        ws = ws / jnp.sum(ws)
        wf = wf_ref[...]
        wf = jnp.exp(wf - jnp.max(wf))
        wf = wf / jnp.sum(wf)
        e1 = ws[0] * xs_ref[0] + ws[1] * xs_ref[1] + ws[2] * xs_ref[2]
        e2 = wf[0] * xf_ref[0] + wf[1] * xf_ref[1] + wf[2] * xf_ref[2]
        lat, al = _attn_core(e1, e2, w_ref[...], u_ref[...])
        lat_ref[...] = lat
        al_ref[...] = al

    return pl.pallas_call(
        body,
        grid=(_NBLK,),
        in_specs=[
            pl.BlockSpec((3, BLK, H), lambda i: (0, i, 0)),
            pl.BlockSpec((3, BLK, H), lambda i: (0, i, 0)),
            pl.BlockSpec((3,), lambda i: (0,)),
            pl.BlockSpec((3,), lambda i: (0,)),
            pl.BlockSpec((H, H), lambda i: (0, 0)),
            pl.BlockSpec((H, 1), lambda i: (0, 0)),
        ],
        out_specs=[
            pl.BlockSpec((BLK, H), lambda i: (i, 0)),
            pl.BlockSpec((BLK, 2), lambda i: (i, 0)),
        ],
        out_shape=[
            jax.ShapeDtypeStruct((N, H), jnp.float32),
            jax.ShapeDtypeStruct((N, 2), jnp.float32),
        ],
    )(xs_sp, xs_fe, wt_sp, wt_fe, w_om, u_om)


def _attn_pair(e1, e2, w_om, u_om):
    H = e1.shape[1]

    def body(e1_ref, e2_ref, w_ref, u_ref, lat_ref, al_ref):
        lat, al = _attn_core(e1_ref[...], e2_ref[...], w_ref[...], u_ref[...])
        lat_ref[...] = lat
        al_ref[...] = al

    return pl.pallas_call(
        body,
        grid=(_NBLK,),
        in_specs=[
            pl.BlockSpec((BLK, H), lambda i: (i, 0)),
            pl.BlockSpec((BLK, H), lambda i: (i, 0)),
            pl.BlockSpec((H, H), lambda i: (0, 0)),
            pl.BlockSpec((H, 1), lambda i: (0, 0)),
        ],
        out_specs=[
            pl.BlockSpec((BLK, H), lambda i: (i, 0)),
            pl.BlockSpec((BLK, 2), lambda i: (i, 0)),
        ],
        out_shape=[
            jax.ShapeDtypeStruct((N, H), jnp.float32),
            jax.ShapeDtypeStruct((N, 2), jnp.float32),
        ],
    )(e1, e2, w_om, u_om)


def kernel(edge_spatial_omics1, edge_feature_omics1, feat_omics1,
           edge_spatial_omics2, edge_feature_omics2, feat_omics2,
           enc1, enc2, dec1, dec2, a1_w, a1_u, a2_w, a2_u, ac_w, ac_u,
           wt1, wt2, wt3, wt4):
    def pad(e):
        src = jnp.concatenate(
            [e[0].astype(jnp.int32), jnp.zeros((EPAD - E,), jnp.int32)])
        dst = jnp.concatenate(
            [e[1].astype(jnp.int32), jnp.full((EPAD - E,), N, jnp.int32)])
        return src, dst

    s1, d1 = pad(edge_spatial_omics1)
    s2, d2 = pad(edge_feature_omics1)
    s3, d3 = pad(edge_spatial_omics2)
    s4, d4 = pad(edge_feature_omics2)
    dst4_c = jnp.concatenate([d1, d2 + NP, d3, d4 + NP])
    dst4_64 = dst4_c.reshape(-1, 64)
    src4_2in = jnp.concatenate([s1, s2, s3 + N, s4 + N]).reshape(-1, 64)
    src4_4in = jnp.concatenate(
        [s1, s2 + N, s3 + 2 * N, s4 + 3 * N]).reshape(-1, 64)
    dst2_c = jnp.concatenate([d1, d3])
    dst2_64 = dst2_c.reshape(-1, 64)
    dst2_32 = dst2_c.reshape(-1, 32)
    src2_1in = jnp.concatenate([s1, s3]).reshape(-1, 64)
    src2_2in = jnp.concatenate([s1, s3 + N]).reshape(-1, 32)

    ones = jnp.ones((64, F_DEG), jnp.float32)

    W1 = jnp.stack([enc1[0][0], enc2[0][0]])
    y = _matmul(jnp.stack([feat_omics1, feat_omics2]), W1)
    msg, deg4 = _segsum(64, 4, 64, True)(
        y.reshape(2 * N, 64), src4_2in, dst4_64, ones)
    deg4 = deg4.reshape(4, NP, F_DEG)
    deg_dec = jnp.stack([deg4[0], deg4[2]])
    b1 = jnp.stack([enc1[0][1], enc1[0][1], enc2[0][1], enc2[0][1]])
    x = _combine(msg.reshape(4, NP, 64), y, 2, deg4, b1)
    hcell = [x]
    for l in (1, 2):
        Wl = jnp.stack([enc1[l][0], enc1[l][0], enc2[l][0], enc2[l][0]])
        bl = jnp.stack([enc1[l][1], enc1[l][1], enc2[l][1], enc2[l][1]])
        y = _matmul(x, Wl)
        msg = _segsum(64, 4, 64)(y.reshape(4 * N, 64), src4_4in, dst4_64)
        x = _combine(msg.reshape(4, NP, 64), y, 1, deg4, bl)
        hcell.append(x)

    xs_sp1 = jnp.stack([h[0] for h in hcell])
    xs_fe1 = jnp.stack([h[1] for h in hcell])
    xs_sp2 = jnp.stack([h[2] for h in hcell])
    xs_fe2 = jnp.stack([h[3] for h in hcell])

    lat1, _ = _attn_prop(xs_sp1, xs_fe1, wt1, wt2, a1_w, a1_u)
    lat2, _ = _attn_prop(xs_sp2, xs_fe2, wt3, wt4, a2_w, a2_u)
    combined, alpha_cross = _attn_pair(lat1, lat2, ac_w, ac_u)

    msg = _segsum(64, 2, 64)(combined, src2_1in, dst2_64).reshape(2, NP, 64)
    h = _combine(msg, combined.reshape(1, N, 64), 2, deg_dec, None)
    Wd1 = jnp.stack([dec1[0][0], dec2[0][0]])
    bd1 = jnp.stack([dec1[0][1], dec2[0][1]])
    xd = _matmul(h, Wd1, bd1)
    for l in (1, 2):
        Wdl = jnp.stack([dec1[l][0], dec2[l][0]])
        bdl = jnp.stack([dec1[l][1], dec2[l][1]])
        yd = _matmul(xd, Wdl)
        msg = _segsum(128, 2, 32)(yd.reshape(2 * N, 128), src2_2in, dst2_32)
        xd = _combine(msg.reshape(2, NP, 128), yd, 1, deg_dec, bdl)

    return (lat1, lat2, combined, xd[0], xd[1], alpha_cross)


# speedup vs baseline: 1.0482x; 1.0105x over previous
import functools

import jax
import jax.numpy as jnp
from jax import lax
from jax.experimental import pallas as pl
from jax.experimental.pallas import tpu as pltpu
from jax.experimental.pallas import tpu_sc as plsc

N = 10000
E = 320000
NP = 10112
STRIPE = NP // 16
EPAD = 327680
C = 128
NC, NS = 2, 16
F_DEG = 16
BLK = 512


def _mesh():
    return plsc.VectorSubcoreMesh(
        core_axis_name="c", subcore_axis_name="s", num_cores=NC, num_subcores=NS
    )


@functools.cache
def _segsum(F, b_out, Cg, with_deg=False):
    b_pc = b_out // NC
    if with_deg:
        K = 2
    elif b_out == 2 and F == 64:
        K = 8
    else:
        K = 5
    R = b_out * EPAD // Cg
    rpt = R // 32
    ngroups = rpt // K
    nfull, rem = STRIPE // Cg, STRIPE % Cg
    nwrit = nfull + (1 if rem else 0)

    def body(*refs):
        if with_deg:
            (x_hbm, src_hbm, dst_hbm, ones_hbm, out_hbm, deg_hbm,
             acc, acc_deg, isrc, idst, bufs, ones, gsem, ssem, isem) = refs
        else:
            (x_hbm, src_hbm, dst_hbm, out_hbm,
             acc, isrc, idst, bufs, gsem, ssem, isem) = refs
        cid = lax.axis_index("c")
        sid = lax.axis_index("s")
        wid = cid * NS + sid

        def zrow(r, _):
            for j in range(F // 16):
                bufs[0, r, pl.ds(j * 16, 16)] = jnp.zeros((16,), jnp.float32)
            return 0

        lax.fori_loop(0, Cg, zrow, 0)
        if with_deg:
            def zdrow(r, _):
                ones[r, :] = jnp.zeros((16,), jnp.float32)
                return 0

            lax.fori_loop(0, Cg, zdrow, 0)
        zd = []
        for l in range(b_pc):
            base = l * NP + sid * STRIPE
            for k in range(nwrit):
                cnt = Cg if k < nfull else rem
                sl = pl.ds(pl.multiple_of(base + k * Cg, 8), cnt)
                zd.append(pltpu.async_copy(
                    bufs.at[0, pl.ds(0, cnt)], acc.at[sl], isem))
                if with_deg:
                    zd.append(pltpu.async_copy(
                        ones.at[pl.ds(0, cnt)], acc_deg.at[sl], isem))
        for d in zd:
            d.wait()
        if with_deg:
            pltpu.sync_copy(ones_hbm, ones)
        plsc.subcore_barrier()

        row0 = wid * rpt

        def idx_rows(g):
            return pl.ds(pl.multiple_of(row0 + g * K, K), K)

        pltpu.sync_copy(src_hbm.at[idx_rows(0)], isrc.at[0])
        pltpu.sync_copy(dst_hbm.at[idx_rows(0)], idst.at[0])
        for k in range(K):
            pltpu.async_copy(x_hbm.at[isrc.at[0, k]], bufs.at[k], gsem)
        pltpu.async_copy(src_hbm.at[idx_rows(1)], isrc.at[1], isem)
        pltpu.async_copy(dst_hbm.at[idx_rows(1)], idst.at[1], isem)

        def quad(i, _):
            for j in range(4):
                S = j % 2
                T = 1 - S
                I = j
                g = 4 * i + j
                for k in range(K):
                    pltpu.make_async_copy(
                        x_hbm.at[pl.ds(0, Cg)], bufs.at[S * K + k], gsem).wait()
                for k in range(K):
                    pltpu.async_copy(bufs.at[S * K + k],
                                     acc.at[idst.at[I, k]], ssem, add=True)
                    if with_deg:
                        pltpu.async_copy(ones, acc_deg.at[idst.at[I, k]],
                                         ssem, add=True)

                @pl.when(g + 1 < ngroups)
                def _():
                    pltpu.make_async_copy(
                        src_hbm.at[pl.ds(0, K)], isrc.at[(j + 1) % 4], isem).wait()
                    pltpu.make_async_copy(
                        dst_hbm.at[pl.ds(0, K)], idst.at[(j + 1) % 4], isem).wait()

                @pl.when(jnp.logical_and(g >= 1, g + 1 < ngroups))
                def _():
                    for k in range(K):
                        pltpu.make_async_copy(
                            x_hbm.at[pl.ds(0, Cg)], bufs.at[T * K + k], ssem).wait()
                        if with_deg:
                            pltpu.make_async_copy(
                                ones_hbm, ones, ssem).wait()

                @pl.when(g + 1 < ngroups)
                def _():
                    for k in range(K):
                        pltpu.async_copy(x_hbm.at[isrc.at[(j + 1) % 4, k]],
                                         bufs.at[T * K + k], gsem)

                @pl.when(g + 2 < ngroups)
                def _():
                    pltpu.async_copy(src_hbm.at[idx_rows(g + 2)],
                                     isrc.at[(j + 2) % 4], isem)
                    pltpu.async_copy(dst_hbm.at[idx_rows(g + 2)],
                                     idst.at[(j + 2) % 4], isem)
            return 0

        lax.fori_loop(0, ngroups // 4, quad, 0)
        for _ in range(2 * K):
            pltpu.make_async_copy(
                x_hbm.at[pl.ds(0, Cg)], bufs.at[0], ssem).wait()
            if with_deg:
                pltpu.make_async_copy(ones_hbm, ones, ssem).wait()
        plsc.subcore_barrier()

        wd = []
        for l in range(b_pc):
            sbase = l * NP + sid * STRIPE
            obase = (cid * b_pc + l) * NP + sid * STRIPE
            for k in range(nwrit):
                cnt = Cg if k < nfull else rem
                ssl = pl.ds(pl.multiple_of(sbase + k * Cg, 8), cnt)
                osl = pl.ds(pl.multiple_of(obase + k * Cg, 8), cnt)
                wd.append(pltpu.async_copy(acc.at[ssl], out_hbm.at[osl], isem))
                if with_deg:
                    wd.append(pltpu.async_copy(
                        acc_deg.at[ssl], deg_hbm.at[osl], isem))
        for d in wd:
            d.wait()

    out_type = jax.ShapeDtypeStruct((b_out * NP, F), jnp.float32)
    scratch = [
        pltpu.VMEM_SHARED((b_pc * NP, F), jnp.float32),
        pltpu.VMEM((4, K, Cg), jnp.int32),
        pltpu.VMEM((4, K, Cg), jnp.int32),
        pltpu.VMEM((2 * K, Cg, F), jnp.float32),
        pltpu.SemaphoreType.DMA,
        pltpu.SemaphoreType.DMA,
        pltpu.SemaphoreType.DMA,
    ]
    if with_deg:
        out_type = [out_type,
                    jax.ShapeDtypeStruct((b_out * NP, F_DEG), jnp.float32)]
        scratch.insert(1, pltpu.VMEM_SHARED((b_pc * NP, F_DEG), jnp.float32))
        scratch.insert(5, pltpu.VMEM((Cg, F_DEG), jnp.float32))
    return pl.kernel(
        body,
        out_type=out_type,
        mesh=_mesh(),
        compiler_params=pltpu.CompilerParams(use_tc_tiling_on_sc=False),
        scratch_types=scratch,
    )


_NBLK = pl.cdiv(N, BLK)


def _matmul(x, w, bias=None):
    B, n, Fi = x.shape
    Fo = w.shape[2]

    def body(x_ref, w_ref, *rest):
        if bias is not None:
            b_ref, o_ref = rest
        else:
            (o_ref,) = rest
        r = jnp.dot(x_ref[0], w_ref[0], preferred_element_type=jnp.float32)
        if bias is not None:
            r = r + b_ref[0]
        o_ref[0] = r

    in_specs = [
        pl.BlockSpec((1, BLK, Fi), lambda b, i: (b, i, 0)),
        pl.BlockSpec((1, Fi, Fo), lambda b, i: (b, 0, 0)),
    ]
    args = [x, w]
    if bias is not None:
        in_specs.append(pl.BlockSpec((1, 1, Fo), lambda b, i: (b, 0, 0)))
        args.append(bias.reshape(B, 1, Fo))
    return pl.pallas_call(
        body,
        grid=(B, _NBLK),
        in_specs=in_specs,
        out_specs=pl.BlockSpec((1, BLK, Fo), lambda b, i: (b, i, 0)),
        out_shape=jax.ShapeDtypeStruct((B, n, Fo), jnp.float32),
    )(*args)


def _combine(msg, y, dup, deg, bias):
    B, _, F = msg.shape

    def body(m_ref, y_ref, d_ref, *rest):
        if bias is not None:
            b_ref, o_ref = rest
        else:
            (o_ref,) = rest
        inv = 1.0 / (d_ref[0][:, 0:1] + 1.0)
        r = (m_ref[0] + y_ref[0]) * inv
        if bias is not None:
            r = r + b_ref[0]
        o_ref[0] = r

    in_specs = [
        pl.BlockSpec((1, BLK, F), lambda b, i: (b, i, 0)),
        pl.BlockSpec((1, BLK, F), lambda b, i: (b // dup, i, 0)),
        pl.BlockSpec((1, BLK, F_DEG), lambda b, i: (b, i, 0)),
    ]
    args = [msg, y, deg]
    if bias is not None:
        in_specs.append(pl.BlockSpec((1, 1, F), lambda b, i: (b, 0, 0)))
        args.append(bias.reshape(B, 1, F))
    return pl.pallas_call(
        body,
        grid=(B, _NBLK),
        in_specs=in_specs,
        out_specs=pl.BlockSpec((1, BLK, F), lambda b, i: (b, i, 0)),
        out_shape=jax.ShapeDtypeStruct((B, N, F), jnp.float32),
    )(*args)


def _attn_core(e1, e2, w_om, u_om):
    v1 = jnp.tanh(jnp.dot(e1, w_om, preferred_element_type=jnp.float32))
    u1 = jnp.dot(v1, u_om, preferred_element_type=jnp.float32)
    v2 = jnp.tanh(jnp.dot(e2, w_om, preferred_element_type=jnp.float32))
    u2 = jnp.dot(v2, u_om, preferred_element_type=jnp.float32)
    m = jnp.maximum(u1, u2)
    a1 = jnp.exp(u1 - m)
    a2 = jnp.exp(u2 - m)
    s = a1 + a2
    a1 = a1 / s
    a2 = a2 / s
    return a1 * e1 + a2 * e2, jnp.concatenate([a1, a2], axis=1)


def _attn_prop(xs_sp, xs_fe, wt_sp, wt_fe, w_om, u_om):
    H = xs_sp.shape[2]

    def body(xs_ref, xf_ref, ws_ref, wf_ref, w_ref, u_ref, lat_ref, al_ref):
        ws = ws_ref[...]
        ws = jnp.exp(ws - jnp.max(ws))
        ws = ws / jnp.sum(ws)
        wf = wf_ref[...]
        wf = jnp.exp(wf - jnp.max(wf))
        wf = wf / jnp.sum(wf)
        e1 = ws[0] * xs_ref[0] + ws[1] * xs_ref[1] + ws[2] * xs_ref[2]
        e2 = wf[0] * xf_ref[0] + wf[1] * xf_ref[1] + wf[2] * xf_ref[2]
        lat, al = _attn_core(e1, e2, w_ref[...], u_ref[...])
        lat_ref[...] = lat
        al_ref[...] = al

    return pl.pallas_call(
        body,
        grid=(_NBLK,),
        in_specs=[
            pl.BlockSpec((3, BLK, H), lambda i: (0, i, 0)),
            pl.BlockSpec((3, BLK, H), lambda i: (0, i, 0)),
            pl.BlockSpec((3,), lambda i: (0,)),
            pl.BlockSpec((3,), lambda i: (0,)),
            pl.BlockSpec((H, H), lambda i: (0, 0)),
            pl.BlockSpec((H, 1), lambda i: (0, 0)),
        ],
        out_specs=[
            pl.BlockSpec((BLK, H), lambda i: (i, 0)),
            pl.BlockSpec((BLK, 2), lambda i: (i, 0)),
        ],
        out_shape=[
            jax.ShapeDtypeStruct((N, H), jnp.float32),
            jax.ShapeDtypeStruct((N, 2), jnp.float32),
        ],
    )(xs_sp, xs_fe, wt_sp, wt_fe, w_om, u_om)


def _attn_pair(e1, e2, w_om, u_om):
    H = e1.shape[1]

    def body(e1_ref, e2_ref, w_ref, u_ref, lat_ref, al_ref):
        lat, al = _attn_core(e1_ref[...], e2_ref[...], w_ref[...], u_ref[...])
        lat_ref[...] = lat
        al_ref[...] = al

    return pl.pallas_call(
        body,
        grid=(_NBLK,),
        in_specs=[
            pl.BlockSpec((BLK, H), lambda i: (i, 0)),
            pl.BlockSpec((BLK, H), lambda i: (i, 0)),
            pl.BlockSpec((H, H), lambda i: (0, 0)),
            pl.BlockSpec((H, 1), lambda i: (0, 0)),
        ],
        out_specs=[
            pl.BlockSpec((BLK, H), lambda i: (i, 0)),
            pl.BlockSpec((BLK, 2), lambda i: (i, 0)),
        ],
        out_shape=[
            jax.ShapeDtypeStruct((N, H), jnp.float32),
            jax.ShapeDtypeStruct((N, 2), jnp.float32),
        ],
    )(e1, e2, w_om, u_om)


def kernel(edge_spatial_omics1, edge_feature_omics1, feat_omics1,
           edge_spatial_omics2, edge_feature_omics2, feat_omics2,
           enc1, enc2, dec1, dec2, a1_w, a1_u, a2_w, a2_u, ac_w, ac_u,
           wt1, wt2, wt3, wt4):
    def pad(e):
        src = jnp.concatenate(
            [e[0].astype(jnp.int32), jnp.zeros((EPAD - E,), jnp.int32)])
        dst = jnp.concatenate(
            [e[1].astype(jnp.int32), jnp.full((EPAD - E,), N, jnp.int32)])
        return src, dst

    s1, d1 = pad(edge_spatial_omics1)
    s2, d2 = pad(edge_feature_omics1)
    s3, d3 = pad(edge_spatial_omics2)
    s4, d4 = pad(edge_feature_omics2)
    dst4_c = jnp.concatenate([d1, d2 + NP, d3, d4 + NP])
    dst4_64 = dst4_c.reshape(-1, 64)
    src4_2in = jnp.concatenate([s1, s2, s3 + N, s4 + N]).reshape(-1, 64)
    src4_4in = jnp.concatenate(
        [s1, s2 + N, s3 + 2 * N, s4 + 3 * N]).reshape(-1, 64)
    dst2_c = jnp.concatenate([d1, d3])
    dst2_64 = dst2_c.reshape(-1, 64)
    dst2_32 = dst2_c.reshape(-1, 32)
    src2_1in = jnp.concatenate([s1, s3]).reshape(-1, 64)
    src2_2in = jnp.concatenate([s1, s3 + N]).reshape(-1, 32)

    ones = jnp.ones((64, F_DEG), jnp.float32)

    W1 = jnp.stack([enc1[0][0], enc2[0][0]])
    y = _matmul(jnp.stack([feat_omics1, feat_omics2]), W1)
    msg, deg4 = _segsum(64, 4, 64, True)(
        y.reshape(2 * N, 64), src4_2in, dst4_64, ones)
    deg4 = deg4.reshape(4, NP, F_DEG)
    deg_dec = jnp.stack([deg4[0], deg4[2]])
    b1 = jnp.stack([enc1[0][1], enc1[0][1], enc2[0][1], enc2[0][1]])
    x = _combine(msg.reshape(4, NP, 64), y, 2, deg4, b1)
    hcell = [x]
    for l in (1, 2):
        Wl = jnp.stack([enc1[l][0], enc1[l][0], enc2[l][0], enc2[l][0]])
        bl = jnp.stack([enc1[l][1], enc1[l][1], enc2[l][1], enc2[l][1]])
        y = _matmul(x, Wl)
        msg = _segsum(64, 4, 64)(y.reshape(4 * N, 64), src4_4in, dst4_64)
        x = _combine(msg.reshape(4, NP, 64), y, 1, deg4, bl)
        hcell.append(x)

    xs_sp1 = jnp.stack([h[0] for h in hcell])
    xs_fe1 = jnp.stack([h[1] for h in hcell])
    xs_sp2 = jnp.stack([h[2] for h in hcell])
    xs_fe2 = jnp.stack([h[3] for h in hcell])

    lat1, _ = _attn_prop(xs_sp1, xs_fe1, wt1, wt2, a1_w, a1_u)
    lat2, _ = _attn_prop(xs_sp2, xs_fe2, wt3, wt4, a2_w, a2_u)
    combined, alpha_cross = _attn_pair(lat1, lat2, ac_w, ac_u)

    msg = _segsum(64, 2, 64)(combined, src2_1in, dst2_64).reshape(2, NP, 64)
    h = _combine(msg, combined.reshape(1, N, 64), 2, deg_dec, None)
    Wd1 = jnp.stack([dec1[0][0], dec2[0][0]])
    bd1 = jnp.stack([dec1[0][1], dec2[0][1]])
    xd = _matmul(h, Wd1, bd1)
    for l in (1, 2):
        Wdl = jnp.stack([dec1[l][0], dec2[l][0]])
        bdl = jnp.stack([dec1[l][1], dec2[l][1]])
        yd = _matmul(xd, Wdl)
        msg = _segsum(128, 2, 32)(yd.reshape(2 * N, 128), src2_2in, dst2_32)
        xd = _combine(msg.reshape(2, NP, 128), yd, 1, deg_dec, bdl)

    return (lat1, lat2, combined, xd[0], xd[1], alpha_cross)


# speedup vs baseline: 1.0483x; 1.0001x over previous
import functools

import jax
import jax.numpy as jnp
from jax import lax
from jax.experimental import pallas as pl
from jax.experimental.pallas import tpu as pltpu
from jax.experimental.pallas import tpu_sc as plsc

N = 10000
E = 320000
NP = 10112
STRIPE = NP // 16
EPAD = 327680
C = 128
NC, NS = 2, 16
F_DEG = 16
BLK = 512


def _mesh():
    return plsc.VectorSubcoreMesh(
        core_axis_name="c", subcore_axis_name="s", num_cores=NC, num_subcores=NS
    )


@functools.cache
def _segsum(F, b_out, Cg, with_deg=False):
    b_pc = b_out // NC
    if with_deg:
        K = 4
    elif b_out == 2 and F == 64:
        K = 10
    else:
        K = 5
    R = b_out * EPAD // Cg
    rpt = R // 32
    ngroups = rpt // K
    nfull, rem = STRIPE // Cg, STRIPE % Cg
    nwrit = nfull + (1 if rem else 0)

    def body(*refs):
        if with_deg:
            (x_hbm, src_hbm, dst_hbm, ones_hbm, out_hbm, deg_hbm,
             acc, acc_deg, isrc, idst, bufs, ones, gsem, ssem, isem) = refs
        else:
            (x_hbm, src_hbm, dst_hbm, out_hbm,
             acc, isrc, idst, bufs, gsem, ssem, isem) = refs
        cid = lax.axis_index("c")
        sid = lax.axis_index("s")
        wid = cid * NS + sid

        def zrow(r, _):
            for j in range(F // 16):
                bufs[0, r, pl.ds(j * 16, 16)] = jnp.zeros((16,), jnp.float32)
            return 0

        lax.fori_loop(0, Cg, zrow, 0)
        if with_deg:
            def zdrow(r, _):
                ones[r, :] = jnp.zeros((16,), jnp.float32)
                return 0

            lax.fori_loop(0, Cg, zdrow, 0)
        zd = []
        for l in range(b_pc):
            base = l * NP + sid * STRIPE
            for k in range(nwrit):
                cnt = Cg if k < nfull else rem
                sl = pl.ds(pl.multiple_of(base + k * Cg, 8), cnt)
                zd.append(pltpu.async_copy(
                    bufs.at[0, pl.ds(0, cnt)], acc.at[sl], isem))
                if with_deg:
                    zd.append(pltpu.async_copy(
                        ones.at[pl.ds(0, cnt)], acc_deg.at[sl], isem))
        for d in zd:
            d.wait()
        if with_deg:
            pltpu.sync_copy(ones_hbm, ones)
        plsc.subcore_barrier()

        row0 = wid * rpt

        def idx_rows(g):
            return pl.ds(pl.multiple_of(row0 + g * K, K), K)

        pltpu.sync_copy(src_hbm.at[idx_rows(0)], isrc.at[0])
        pltpu.sync_copy(dst_hbm.at[idx_rows(0)], idst.at[0])
        for k in range(K):
            pltpu.async_copy(x_hbm.at[isrc.at[0, k]], bufs.at[k], gsem)
        pltpu.async_copy(src_hbm.at[idx_rows(1)], isrc.at[1], isem)
        pltpu.async_copy(dst_hbm.at[idx_rows(1)], idst.at[1], isem)

        def quad(i, _):
            for j in range(4):
                S = j % 2
                T = 1 - S
                I = j
                g = 4 * i + j
                for k in range(K):
                    pltpu.make_async_copy(
                        x_hbm.at[pl.ds(0, Cg)], bufs.at[S * K + k], gsem).wait()
                for k in range(K):
                    pltpu.async_copy(bufs.at[S * K + k],
                                     acc.at[idst.at[I, k]], ssem, add=True)
                    if with_deg:
                        pltpu.async_copy(ones, acc_deg.at[idst.at[I, k]],
                                         ssem, add=True)

                @pl.when(g + 1 < ngroups)
                def _():
                    pltpu.make_async_copy(
                        src_hbm.at[pl.ds(0, K)], isrc.at[(j + 1) % 4], isem).wait()
                    pltpu.make_async_copy(
                        dst_hbm.at[pl.ds(0, K)], idst.at[(j + 1) % 4], isem).wait()

                @pl.when(jnp.logical_and(g >= 1, g + 1 < ngroups))
                def _():
                    for k in range(K):
                        pltpu.make_async_copy(
                            x_hbm.at[pl.ds(0, Cg)], bufs.at[T * K + k], ssem).wait()
                        if with_deg:
                            pltpu.make_async_copy(
                                ones_hbm, ones, ssem).wait()

                @pl.when(g + 1 < ngroups)
                def _():
                    for k in range(K):
                        pltpu.async_copy(x_hbm.at[isrc.at[(j + 1) % 4, k]],
                                         bufs.at[T * K + k], gsem)

                @pl.when(g + 2 < ngroups)
                def _():
                    pltpu.async_copy(src_hbm.at[idx_rows(g + 2)],
                                     isrc.at[(j + 2) % 4], isem)
                    pltpu.async_copy(dst_hbm.at[idx_rows(g + 2)],
                                     idst.at[(j + 2) % 4], isem)
            return 0

        lax.fori_loop(0, ngroups // 4, quad, 0)
        for _ in range(2 * K):
            pltpu.make_async_copy(
                x_hbm.at[pl.ds(0, Cg)], bufs.at[0], ssem).wait()
            if with_deg:
                pltpu.make_async_copy(ones_hbm, ones, ssem).wait()
        plsc.subcore_barrier()

        wd = []
        for l in range(b_pc):
            sbase = l * NP + sid * STRIPE
            obase = (cid * b_pc + l) * NP + sid * STRIPE
            for k in range(nwrit):
                cnt = Cg if k < nfull else rem
                ssl = pl.ds(pl.multiple_of(sbase + k * Cg, 8), cnt)
                osl = pl.ds(pl.multiple_of(obase + k * Cg, 8), cnt)
                wd.append(pltpu.async_copy(acc.at[ssl], out_hbm.at[osl], isem))
                if with_deg:
                    wd.append(pltpu.async_copy(
                        acc_deg.at[ssl], deg_hbm.at[osl], isem))
        for d in wd:
            d.wait()

    out_type = jax.ShapeDtypeStruct((b_out * NP, F), jnp.float32)
    scratch = [
        pltpu.VMEM_SHARED((b_pc * NP, F), jnp.float32),
        pltpu.VMEM((4, K, Cg), jnp.int32),
        pltpu.VMEM((4, K, Cg), jnp.int32),
        pltpu.VMEM((2 * K, Cg, F), jnp.float32),
        pltpu.SemaphoreType.DMA,
        pltpu.SemaphoreType.DMA,
        pltpu.SemaphoreType.DMA,
    ]
    if with_deg:
        out_type = [out_type,
                    jax.ShapeDtypeStruct((b_out * NP, F_DEG), jnp.float32)]
        scratch.insert(1, pltpu.VMEM_SHARED((b_pc * NP, F_DEG), jnp.float32))
        scratch.insert(5, pltpu.VMEM((Cg, F_DEG), jnp.float32))
    return pl.kernel(
        body,
        out_type=out_type,
        mesh=_mesh(),
        compiler_params=pltpu.CompilerParams(use_tc_tiling_on_sc=False),
        scratch_types=scratch,
    )


_NBLK = pl.cdiv(N, BLK)


def _matmul(x, w, bias=None):
    B, n, Fi = x.shape
    Fo = w.shape[2]

    def body(x_ref, w_ref, *rest):
        if bias is not None:
            b_ref, o_ref = rest
        else:
            (o_ref,) = rest
        r = jnp.dot(x_ref[0], w_ref[0], preferred_element_type=jnp.float32)
        if bias is not None:
            r = r + b_ref[0]
        o_ref[0] = r

    in_specs = [
        pl.BlockSpec((1, BLK, Fi), lambda b, i: (b, i, 0)),
        pl.BlockSpec((1, Fi, Fo), lambda b, i: (b, 0, 0)),
    ]
    args = [x, w]
    if bias is not None:
        in_specs.append(pl.BlockSpec((1, 1, Fo), lambda b, i: (b, 0, 0)))
        args.append(bias.reshape(B, 1, Fo))
    return pl.pallas_call(
        body,
        grid=(B, _NBLK),
        in_specs=in_specs,
        out_specs=pl.BlockSpec((1, BLK, Fo), lambda b, i: (b, i, 0)),
        out_shape=jax.ShapeDtypeStruct((B, n, Fo), jnp.float32),
    )(*args)


def _combine(msg, y, dup, deg, bias):
    B, _, F = msg.shape

    def body(m_ref, y_ref, d_ref, *rest):
        if bias is not None:
            b_ref, o_ref = rest
        else:
            (o_ref,) = rest
        inv = 1.0 / (d_ref[0][:, 0:1] + 1.0)
        r = (m_ref[0] + y_ref[0]) * inv
        if bias is not None:
            r = r + b_ref[0]
        o_ref[0] = r

    in_specs = [
        pl.BlockSpec((1, BLK, F), lambda b, i: (b, i, 0)),
        pl.BlockSpec((1, BLK, F), lambda b, i: (b // dup, i, 0)),
        pl.BlockSpec((1, BLK, F_DEG), lambda b, i: (b, i, 0)),
    ]
    args = [msg, y, deg]
    if bias is not None:
        in_specs.append(pl.BlockSpec((1, 1, F), lambda b, i: (b, 0, 0)))
        args.append(bias.reshape(B, 1, F))
    return pl.pallas_call(
        body,
        grid=(B, _NBLK),
        in_specs=in_specs,
        out_specs=pl.BlockSpec((1, BLK, F), lambda b, i: (b, i, 0)),
        out_shape=jax.ShapeDtypeStruct((B, N, F), jnp.float32),
    )(*args)


def _attn_core(e1, e2, w_om, u_om):
    v1 = jnp.tanh(jnp.dot(e1, w_om, preferred_element_type=jnp.float32))
    u1 = jnp.dot(v1, u_om, preferred_element_type=jnp.float32)
    v2 = jnp.tanh(jnp.dot(e2, w_om, preferred_element_type=jnp.float32))
    u2 = jnp.dot(v2, u_om, preferred_element_type=jnp.float32)
    m = jnp.maximum(u1, u2)
    a1 = jnp.exp(u1 - m)
    a2 = jnp.exp(u2 - m)
    s = a1 + a2
    a1 = a1 / s
    a2 = a2 / s
    return a1 * e1 + a2 * e2, jnp.concatenate([a1, a2], axis=1)


def _attn_prop(xs_sp, xs_fe, wt_sp, wt_fe, w_om, u_om):
    H = xs_sp.shape[2]

    def body(xs_ref, xf_ref, ws_ref, wf_ref, w_ref, u_ref, lat_ref, al_ref):
        ws = ws_ref[...]
        ws = jnp.exp(ws - jnp.max(ws))
        ws = ws / jnp.sum(ws)
        wf = wf_ref[...]
        wf = jnp.exp(wf - jnp.max(wf))
        wf = wf / jnp.sum(wf)
        e1 = ws[0] * xs_ref[0] + ws[1] * xs_ref[1] + ws[2] * xs_ref[2]
        e2 = wf[0] * xf_ref[0] + wf[1] * xf_ref[1] + wf[2] * xf_ref[2]
        lat, al = _attn_core(e1, e2, w_ref[...], u_ref[...])
        lat_ref[...] = lat
        al_ref[...] = al

    return pl.pallas_call(
        body,
        grid=(_NBLK,),
        in_specs=[
            pl.BlockSpec((3, BLK, H), lambda i: (0, i, 0)),
            pl.BlockSpec((3, BLK, H), lambda i: (0, i, 0)),
            pl.BlockSpec((3,), lambda i: (0,)),
            pl.BlockSpec((3,), lambda i: (0,)),
            pl.BlockSpec((H, H), lambda i: (0, 0)),
            pl.BlockSpec((H, 1), lambda i: (0, 0)),
        ],
        out_specs=[
            pl.BlockSpec((BLK, H), lambda i: (i, 0)),
            pl.BlockSpec((BLK, 2), lambda i: (i, 0)),
        ],
        out_shape=[
            jax.ShapeDtypeStruct((N, H), jnp.float32),
            jax.ShapeDtypeStruct((N, 2), jnp.float32),
        ],
    )(xs_sp, xs_fe, wt_sp, wt_fe, w_om, u_om)


def _attn_pair(e1, e2, w_om, u_om):
    H = e1.shape[1]

    def body(e1_ref, e2_ref, w_ref, u_ref, lat_ref, al_ref):
        lat, al = _attn_core(e1_ref[...], e2_ref[...], w_ref[...], u_ref[...])
        lat_ref[...] = lat
        al_ref[...] = al

    return pl.pallas_call(
        body,
        grid=(_NBLK,),
        in_specs=[
            pl.BlockSpec((BLK, H), lambda i: (i, 0)),
            pl.BlockSpec((BLK, H), lambda i: (i, 0)),
            pl.BlockSpec((H, H), lambda i: (0, 0)),
            pl.BlockSpec((H, 1), lambda i: (0, 0)),
        ],
        out_specs=[
            pl.BlockSpec((BLK, H), lambda i: (i, 0)),
            pl.BlockSpec((BLK, 2), lambda i: (i, 0)),
        ],
        out_shape=[
            jax.ShapeDtypeStruct((N, H), jnp.float32),
            jax.ShapeDtypeStruct((N, 2), jnp.float32),
        ],
    )(e1, e2, w_om, u_om)


def kernel(edge_spatial_omics1, edge_feature_omics1, feat_omics1,
           edge_spatial_omics2, edge_feature_omics2, feat_omics2,
           enc1, enc2, dec1, dec2, a1_w, a1_u, a2_w, a2_u, ac_w, ac_u,
           wt1, wt2, wt3, wt4):
    def pad(e):
        src = jnp.concatenate(
            [e[0].astype(jnp.int32), jnp.zeros((EPAD - E,), jnp.int32)])
        dst = jnp.concatenate(
            [e[1].astype(jnp.int32), jnp.full((EPAD - E,), N, jnp.int32)])
        return src, dst

    s1, d1 = pad(edge_spatial_omics1)
    s2, d2 = pad(edge_feature_omics1)
    s3, d3 = pad(edge_spatial_omics2)
    s4, d4 = pad(edge_feature_omics2)
    dst4_c = jnp.concatenate([d1, d2 + NP, d3, d4 + NP])
    dst4_64 = dst4_c.reshape(-1, 64)
    src4_2in = jnp.concatenate([s1, s2, s3 + N, s4 + N]).reshape(-1, 64)
    src4_4in = jnp.concatenate(
        [s1, s2 + N, s3 + 2 * N, s4 + 3 * N]).reshape(-1, 64)
    dst2_c = jnp.concatenate([d1, d3])
    dst2_64 = dst2_c.reshape(-1, 64)
    dst2_32 = dst2_c.reshape(-1, 32)
    src2_1in = jnp.concatenate([s1, s3]).reshape(-1, 64)
    src2_2in = jnp.concatenate([s1, s3 + N]).reshape(-1, 32)

    ones = jnp.ones((64, F_DEG), jnp.float32)

    W1 = jnp.stack([enc1[0][0], enc2[0][0]])
    y = _matmul(jnp.stack([feat_omics1, feat_omics2]), W1)
    msg, deg4 = _segsum(64, 4, 32, True)(
        y.reshape(2 * N, 64), src4_2in.reshape(-1, 32),
        dst4_64.reshape(-1, 32), ones[:32])
    deg4 = deg4.reshape(4, NP, F_DEG)
    deg_dec = jnp.stack([deg4[0], deg4[2]])
    b1 = jnp.stack([enc1[0][1], enc1[0][1], enc2[0][1], enc2[0][1]])
    x = _combine(msg.reshape(4, NP, 64), y, 2, deg4, b1)
    hcell = [x]
    for l in (1, 2):
        Wl = jnp.stack([enc1[l][0], enc1[l][0], enc2[l][0], enc2[l][0]])
        bl = jnp.stack([enc1[l][1], enc1[l][1], enc2[l][1], enc2[l][1]])
        y = _matmul(x, Wl)
        msg = _segsum(64, 4, 64)(y.reshape(4 * N, 64), src4_4in, dst4_64)
        x = _combine(msg.reshape(4, NP, 64), y, 1, deg4, bl)
        hcell.append(x)

    xs_sp1 = jnp.stack([h[0] for h in hcell])
    xs_fe1 = jnp.stack([h[1] for h in hcell])
    xs_sp2 = jnp.stack([h[2] for h in hcell])
    xs_fe2 = jnp.stack([h[3] for h in hcell])

    lat1, _ = _attn_prop(xs_sp1, xs_fe1, wt1, wt2, a1_w, a1_u)
    lat2, _ = _attn_prop(xs_sp2, xs_fe2, wt3, wt4, a2_w, a2_u)
    combined, alpha_cross = _attn_pair(lat1, lat2, ac_w, ac_u)

    msg = _segsum(64, 2, 64)(combined, src2_1in, dst2_64).reshape(2, NP, 64)
    h = _combine(msg, combined.reshape(1, N, 64), 2, deg_dec, None)
    Wd1 = jnp.stack([dec1[0][0], dec2[0][0]])
    bd1 = jnp.stack([dec1[0][1], dec2[0][1]])
    xd = _matmul(h, Wd1, bd1)
    for l in (1, 2):
        Wdl = jnp.stack([dec1[l][0], dec2[l][0]])
        bdl = jnp.stack([dec1[l][1], dec2[l][1]])
        yd = _matmul(xd, Wdl)
        msg = _segsum(128, 2, 32)(yd.reshape(2 * N, 128), src2_2in, dst2_32)
        xd = _combine(msg.reshape(2, NP, 128), yd, 1, deg_dec, bdl)

    return (lat1, lat2, combined, xd[0], xd[1], alpha_cross)


# speedup vs baseline: 1.0483x; 1.0000x over previous
import functools

import jax
import jax.numpy as jnp
from jax import lax
from jax.experimental import pallas as pl
from jax.experimental.pallas import tpu as pltpu
from jax.experimental.pallas import tpu_sc as plsc

N = 10000
E = 320000
NP = 10112
STRIPE = NP // 16
EPAD = 327680
C = 128
NC, NS = 2, 16
F_DEG = 16
BLK = 512


def _mesh():
    return plsc.VectorSubcoreMesh(
        core_axis_name="c", subcore_axis_name="s", num_cores=NC, num_subcores=NS
    )


@functools.cache
def _segsum(F, b_out, Cg, with_deg=False):
    b_pc = b_out // NC
    if with_deg:
        K = 2
    elif b_out == 2 and F == 64:
        K = 8
    else:
        K = 5
    R = b_out * EPAD // Cg
    rpt = R // 32
    ngroups = rpt // K
    nfull, rem = STRIPE // Cg, STRIPE % Cg
    nwrit = nfull + (1 if rem else 0)

    def body(*refs):
        if with_deg:
            (x_hbm, src_hbm, dst_hbm, ones_hbm, out_hbm, deg_hbm,
             acc, acc_deg, isrc, idst, bufs, ones, gsem, ssem, isem) = refs
        else:
            (x_hbm, src_hbm, dst_hbm, out_hbm,
             acc, isrc, idst, bufs, gsem, ssem, isem) = refs
        cid = lax.axis_index("c")
        sid = lax.axis_index("s")
        wid = cid * NS + sid

        def zrow(r, _):
            for j in range(F // 16):
                bufs[0, r, pl.ds(j * 16, 16)] = jnp.zeros((16,), jnp.float32)
            return 0

        lax.fori_loop(0, Cg, zrow, 0)
        if with_deg:
            def zdrow(r, _):
                ones[r, :] = jnp.zeros((16,), jnp.float32)
                return 0

            lax.fori_loop(0, Cg, zdrow, 0)
        zd = []
        for l in range(b_pc):
            base = l * NP + sid * STRIPE
            for k in range(nwrit):
                cnt = Cg if k < nfull else rem
                sl = pl.ds(pl.multiple_of(base + k * Cg, 8), cnt)
                zd.append(pltpu.async_copy(
                    bufs.at[0, pl.ds(0, cnt)], acc.at[sl], isem))
                if with_deg:
                    zd.append(pltpu.async_copy(
                        ones.at[pl.ds(0, cnt)], acc_deg.at[sl], isem))
        for d in zd:
            d.wait()
        if with_deg:
            pltpu.sync_copy(ones_hbm, ones)
        plsc.subcore_barrier()

        row0 = wid * rpt

        def idx_rows(g):
            return pl.ds(pl.multiple_of(row0 + g * K, K), K)

        pltpu.sync_copy(src_hbm.at[idx_rows(0)], isrc.at[0])
        pltpu.sync_copy(dst_hbm.at[idx_rows(0)], idst.at[0])
        for k in range(K):
            pltpu.async_copy(x_hbm.at[isrc.at[0, k]], bufs.at[k], gsem)
        pltpu.async_copy(src_hbm.at[idx_rows(1)], isrc.at[1], isem)
        pltpu.async_copy(dst_hbm.at[idx_rows(1)], idst.at[1], isem)

        def quad(i, _):
            for j in range(4):
                S = j % 2
                T = 1 - S
                I = j
                g = 4 * i + j
                for k in range(K):
                    pltpu.make_async_copy(
                        x_hbm.at[pl.ds(0, Cg)], bufs.at[S * K + k], gsem).wait()
                for k in range(K):
                    pltpu.async_copy(bufs.at[S * K + k],
                                     acc.at[idst.at[I, k]], ssem, add=True)
                    if with_deg:
                        pltpu.async_copy(ones, acc_deg.at[idst.at[I, k]],
                                         ssem, add=True)

                @pl.when(g + 1 < ngroups)
                def _():
                    pltpu.make_async_copy(
                        src_hbm.at[pl.ds(0, K)], isrc.at[(j + 1) % 4], isem).wait()
                    pltpu.make_async_copy(
                        dst_hbm.at[pl.ds(0, K)], idst.at[(j + 1) % 4], isem).wait()

                @pl.when(jnp.logical_and(g >= 1, g + 1 < ngroups))
                def _():
                    for k in range(K):
                        pltpu.make_async_copy(
                            x_hbm.at[pl.ds(0, Cg)], bufs.at[T * K + k], ssem).wait()
                        if with_deg:
                            pltpu.make_async_copy(
                                ones_hbm, ones, ssem).wait()

                @pl.when(g + 1 < ngroups)
                def _():
                    for k in range(K):
                        pltpu.async_copy(x_hbm.at[isrc.at[(j + 1) % 4, k]],
                                         bufs.at[T * K + k], gsem)

                @pl.when(g + 2 < ngroups)
                def _():
                    pltpu.async_copy(src_hbm.at[idx_rows(g + 2)],
                                     isrc.at[(j + 2) % 4], isem)
                    pltpu.async_copy(dst_hbm.at[idx_rows(g + 2)],
                                     idst.at[(j + 2) % 4], isem)
            return 0

        lax.fori_loop(0, ngroups // 4, quad, 0)
        for _ in range(2 * K):
            pltpu.make_async_copy(
                x_hbm.at[pl.ds(0, Cg)], bufs.at[0], ssem).wait()
            if with_deg:
                pltpu.make_async_copy(ones_hbm, ones, ssem).wait()
        plsc.subcore_barrier()

        wd = []
        for l in range(b_pc):
            sbase = l * NP + sid * STRIPE
            obase = (cid * b_pc + l) * NP + sid * STRIPE
            for k in range(nwrit):
                cnt = Cg if k < nfull else rem
                ssl = pl.ds(pl.multiple_of(sbase + k * Cg, 8), cnt)
                osl = pl.ds(pl.multiple_of(obase + k * Cg, 8), cnt)
                wd.append(pltpu.async_copy(acc.at[ssl], out_hbm.at[osl], isem))
                if with_deg:
                    wd.append(pltpu.async_copy(
                        acc_deg.at[ssl], deg_hbm.at[osl], isem))
        for d in wd:
            d.wait()

    out_type = jax.ShapeDtypeStruct((b_out * NP, F), jnp.float32)
    scratch = [
        pltpu.VMEM_SHARED((b_pc * NP, F), jnp.float32),
        pltpu.VMEM((4, K, Cg), jnp.int32),
        pltpu.VMEM((4, K, Cg), jnp.int32),
        pltpu.VMEM((2 * K, Cg, F), jnp.float32),
        pltpu.SemaphoreType.DMA,
        pltpu.SemaphoreType.DMA,
        pltpu.SemaphoreType.DMA,
    ]
    if with_deg:
        out_type = [out_type,
                    jax.ShapeDtypeStruct((b_out * NP, F_DEG), jnp.float32)]
        scratch.insert(1, pltpu.VMEM_SHARED((b_pc * NP, F_DEG), jnp.float32))
        scratch.insert(5, pltpu.VMEM((Cg, F_DEG), jnp.float32))
    return pl.kernel(
        body,
        out_type=out_type,
        mesh=_mesh(),
        compiler_params=pltpu.CompilerParams(use_tc_tiling_on_sc=False),
        scratch_types=scratch,
    )


_NBLK = pl.cdiv(N, BLK)


def _matmul(x, w, bias=None):
    B, n, Fi = x.shape
    Fo = w.shape[2]

    def body(x_ref, w_ref, *rest):
        if bias is not None:
            b_ref, o_ref = rest
        else:
            (o_ref,) = rest
        r = jnp.dot(x_ref[0], w_ref[0], preferred_element_type=jnp.float32)
        if bias is not None:
            r = r + b_ref[0]
        o_ref[0] = r

    in_specs = [
        pl.BlockSpec((1, BLK, Fi), lambda b, i: (b, i, 0)),
        pl.BlockSpec((1, Fi, Fo), lambda b, i: (b, 0, 0)),
    ]
    args = [x, w]
    if bias is not None:
        in_specs.append(pl.BlockSpec((1, 1, Fo), lambda b, i: (b, 0, 0)))
        args.append(bias.reshape(B, 1, Fo))
    return pl.pallas_call(
        body,
        grid=(B, _NBLK),
        in_specs=in_specs,
        out_specs=pl.BlockSpec((1, BLK, Fo), lambda b, i: (b, i, 0)),
        out_shape=jax.ShapeDtypeStruct((B, n, Fo), jnp.float32),
    )(*args)


def _combine(msg, y, dup, deg, bias):
    B, _, F = msg.shape

    def body(m_ref, y_ref, d_ref, *rest):
        if bias is not None:
            b_ref, o_ref = rest
        else:
            (o_ref,) = rest
        inv = 1.0 / (d_ref[0][:, 0:1] + 1.0)
        r = (m_ref[0] + y_ref[0]) * inv
        if bias is not None:
            r = r + b_ref[0]
        o_ref[0] = r

    in_specs = [
        pl.BlockSpec((1, BLK, F), lambda b, i: (b, i, 0)),
        pl.BlockSpec((1, BLK, F), lambda b, i: (b // dup, i, 0)),
        pl.BlockSpec((1, BLK, F_DEG), lambda b, i: (b, i, 0)),
    ]
    args = [msg, y, deg]
    if bias is not None:
        in_specs.append(pl.BlockSpec((1, 1, F), lambda b, i: (b, 0, 0)))
        args.append(bias.reshape(B, 1, F))
    return pl.pallas_call(
        body,
        grid=(B, _NBLK),
        in_specs=in_specs,
        out_specs=pl.BlockSpec((1, BLK, F), lambda b, i: (b, i, 0)),
        out_shape=jax.ShapeDtypeStruct((B, N, F), jnp.float32),
    )(*args)


def _attn_core(e1, e2, w_om, u_om):
    v1 = jnp.tanh(jnp.dot(e1, w_om, preferred_element_type=jnp.float32))
    u1 = jnp.dot(v1, u_om, preferred_element_type=jnp.float32)
    v2 = jnp.tanh(jnp.dot(e2, w_om, preferred_element_type=jnp.float32))
    u2 = jnp.dot(v2, u_om, preferred_element_type=jnp.float32)
    m = jnp.maximum(u1, u2)
    a1 = jnp.exp(u1 - m)
    a2 = jnp.exp(u2 - m)
    s = a1 + a2
    a1 = a1 / s
    a2 = a2 / s
    return a1 * e1 + a2 * e2, jnp.concatenate([a1, a2], axis=1)


def _attn_prop(xs_sp, xs_fe, wt_sp, wt_fe, w_om, u_om):
    H = xs_sp.shape[2]

    def body(xs_ref, xf_ref, ws_ref, wf_ref, w_ref, u_ref, lat_ref, al_ref):
        ws = ws_ref[...]
        ws = jnp.exp(ws - jnp.max(ws))
        ws = ws / jnp.sum(ws)
        wf = wf_ref[...]
        wf = jnp.exp(wf - jnp.max(wf))
        wf = wf / jnp.sum(wf)
        e1 = ws[0] * xs_ref[0] + ws[1] * xs_ref[1] + ws[2] * xs_ref[2]
        e2 = wf[0] * xf_ref[0] + wf[1] * xf_ref[1] + wf[2] * xf_ref[2]
        lat, al = _attn_core(e1, e2, w_ref[...], u_ref[...])
        lat_ref[...] = lat
        al_ref[...] = al

    return pl.pallas_call(
        body,
        grid=(_NBLK,),
        in_specs=[
            pl.BlockSpec((3, BLK, H), lambda i: (0, i, 0)),
            pl.BlockSpec((3, BLK, H), lambda i: (0, i, 0)),
            pl.BlockSpec((3,), lambda i: (0,)),
            pl.BlockSpec((3,), lambda i: (0,)),
            pl.BlockSpec((H, H), lambda i: (0, 0)),
            pl.BlockSpec((H, 1), lambda i: (0, 0)),
        ],
        out_specs=[
            pl.BlockSpec((BLK, H), lambda i: (i, 0)),
            pl.BlockSpec((BLK, 2), lambda i: (i, 0)),
        ],
        out_shape=[
            jax.ShapeDtypeStruct((N, H), jnp.float32),
            jax.ShapeDtypeStruct((N, 2), jnp.float32),
        ],
    )(xs_sp, xs_fe, wt_sp, wt_fe, w_om, u_om)


def _attn_pair(e1, e2, w_om, u_om):
    H = e1.shape[1]

    def body(e1_ref, e2_ref, w_ref, u_ref, lat_ref, al_ref):
        lat, al = _attn_core(e1_ref[...], e2_ref[...], w_ref[...], u_ref[...])
        lat_ref[...] = lat
        al_ref[...] = al

    return pl.pallas_call(
        body,
        grid=(_NBLK,),
        in_specs=[
            pl.BlockSpec((BLK, H), lambda i: (i, 0)),
            pl.BlockSpec((BLK, H), lambda i: (i, 0)),
            pl.BlockSpec((H, H), lambda i: (0, 0)),
            pl.BlockSpec((H, 1), lambda i: (0, 0)),
        ],
        out_specs=[
            pl.BlockSpec((BLK, H), lambda i: (i, 0)),
            pl.BlockSpec((BLK, 2), lambda i: (i, 0)),
        ],
        out_shape=[
            jax.ShapeDtypeStruct((N, H), jnp.float32),
            jax.ShapeDtypeStruct((N, 2), jnp.float32),
        ],
    )(e1, e2, w_om, u_om)


def kernel(edge_spatial_omics1, edge_feature_omics1, feat_omics1,
           edge_spatial_omics2, edge_feature_omics2, feat_omics2,
           enc1, enc2, dec1, dec2, a1_w, a1_u, a2_w, a2_u, ac_w, ac_u,
           wt1, wt2, wt3, wt4):
    def pad(e):
        src = jnp.concatenate(
            [e[0].astype(jnp.int32), jnp.zeros((EPAD - E,), jnp.int32)])
        dst = jnp.concatenate(
            [e[1].astype(jnp.int32), jnp.full((EPAD - E,), N, jnp.int32)])
        return src, dst

    s1, d1 = pad(edge_spatial_omics1)
    s2, d2 = pad(edge_feature_omics1)
    s3, d3 = pad(edge_spatial_omics2)
    s4, d4 = pad(edge_feature_omics2)
    dst4_c = jnp.concatenate([d1, d2 + NP, d3, d4 + NP])
    dst4_64 = dst4_c.reshape(-1, 64)
    src4_2in = jnp.concatenate([s1, s2, s3 + N, s4 + N]).reshape(-1, 64)
    src4_4in = jnp.concatenate(
        [s1, s2 + N, s3 + 2 * N, s4 + 3 * N]).reshape(-1, 64)
    dst2_c = jnp.concatenate([d1, d3])
    dst2_64 = dst2_c.reshape(-1, 64)
    dst2_32 = dst2_c.reshape(-1, 32)
    src2_1in = jnp.concatenate([s1, s3]).reshape(-1, 64)
    src2_2in = jnp.concatenate([s1, s3 + N]).reshape(-1, 32)

    ones = jnp.ones((64, F_DEG), jnp.float32)

    W1 = jnp.stack([enc1[0][0], enc2[0][0]])
    y = _matmul(jnp.stack([feat_omics1, feat_omics2]), W1)
    msg, deg4 = _segsum(64, 4, 64, True)(
        y.reshape(2 * N, 64), src4_2in, dst4_64, ones)
    deg4 = deg4.reshape(4, NP, F_DEG)
    deg_dec = jnp.stack([deg4[0], deg4[2]])
    b1 = jnp.stack([enc1[0][1], enc1[0][1], enc2[0][1], enc2[0][1]])
    x = _combine(msg.reshape(4, NP, 64), y, 2, deg4, b1)
    hcell = [x]
    for l in (1, 2):
        Wl = jnp.stack([enc1[l][0], enc1[l][0], enc2[l][0], enc2[l][0]])
        bl = jnp.stack([enc1[l][1], enc1[l][1], enc2[l][1], enc2[l][1]])
        y = _matmul(x, Wl)
        msg = _segsum(64, 4, 64)(y.reshape(4 * N, 64), src4_4in, dst4_64)
        x = _combine(msg.reshape(4, NP, 64), y, 1, deg4, bl)
        hcell.append(x)

    xs_sp1 = jnp.stack([h[0] for h in hcell])
    xs_fe1 = jnp.stack([h[1] for h in hcell])
    xs_sp2 = jnp.stack([h[2] for h in hcell])
    xs_fe2 = jnp.stack([h[3] for h in hcell])

    lat1, _ = _attn_prop(xs_sp1, xs_fe1, wt1, wt2, a1_w, a1_u)
    lat2, _ = _attn_prop(xs_sp2, xs_fe2, wt3, wt4, a2_w, a2_u)
    combined, alpha_cross = _attn_pair(lat1, lat2, ac_w, ac_u)

    msg = _segsum(64, 2, 64)(combined, src2_1in, dst2_64).reshape(2, NP, 64)
    h = _combine(msg, combined.reshape(1, N, 64), 2, deg_dec, None)
    Wd1 = jnp.stack([dec1[0][0], dec2[0][0]])
    bd1 = jnp.stack([dec1[0][1], dec2[0][1]])
    xd = _matmul(h, Wd1, bd1)
    for l in (1, 2):
        Wdl = jnp.stack([dec1[l][0], dec2[l][0]])
        bdl = jnp.stack([dec1[l][1], dec2[l][1]])
        yd = _matmul(xd, Wdl)
        msg = _segsum(128, 2, 32)(yd.reshape(2 * N, 128), src2_2in, dst2_32)
        xd = _combine(msg.reshape(2, NP, 128), yd, 1, deg_dec, bdl)

    return (lat1, lat2, combined, xd[0], xd[1], alpha_cross)
